# baseline (speedup 1.0000x reference)
import functools

import jax
import jax.numpy as jnp
from jax.experimental import pallas as pl
from jax.experimental.pallas import tpu as pltpu

_LEAD = 128


def _top2(probs, tile, ne):
    eidx = jax.lax.broadcasted_iota(jnp.int32, (tile, ne), 1)
    p0 = jnp.max(probs, axis=-1, keepdims=True)
    e0 = jnp.min(jnp.where(probs == p0, eidx, ne), axis=-1, keepdims=True)
    masked = jnp.where(eidx == e0, -jnp.inf, probs)
    p1 = jnp.max(masked, axis=-1, keepdims=True)
    e1 = jnp.min(jnp.where(masked == p1, eidx, ne), axis=-1, keepdims=True)
    return p0, e0, p1, e1


def _lead_blocks(probs, c0, tile, ne):
    eidx = jax.lax.broadcasted_iota(jnp.int32, (tile, ne), 1)
    p0, e0, p1, e1 = _top2(probs, tile, ne)
    s = p0 + p1
    p0n = p0 / s
    p1n = p1 / s
    slot1 = jnp.sum(jnp.where(eidx == e1, c0, 0.0), axis=-1,
                    keepdims=True).astype(jnp.int32)
    slotmat = jnp.where(eidx == e0, 0, jnp.where(eidx == e1, slot1, -1))
    valmat = jnp.where(eidx == e0, p0n, jnp.where(eidx == e1, p1n, 0.0))
    siota = jax.lax.broadcasted_iota(jnp.int32, (tile, ne, _LEAD), 2)
    hit = siota == slotmat[:, :, None]
    return hit.astype(jnp.float32), jnp.where(hit, valmat[:, :, None], 0.0), e1


def _router_kernel(x_ref, w1_ref, b1_ref, w2_ref, b2_ref,
                   disp_ref, comb_ref, probs_ref, aux_ref,
                   probs_s, c0_s, snap_s, sums_s, abuf_d, abuf_c,
                   fix_d, fix_c, sem_ad, sem_ac, sem_f,
                   *, tile, tiles, ne, cap, ntok):
    p = pl.program_id(0)
    t = pl.program_id(1)
    eidx = jax.lax.broadcasted_iota(jnp.int32, (tile, ne), 1)

    def _acopy(src, dst_ref, s, step, sem):
        return pltpu.make_async_copy(
            src.at[s], dst_ref.at[pl.ds(step * tile, tile)], sem.at[s])

    @pl.when(p == 0)
    def _phase_a():
        slot = t % 2

        x = x_ref[...]
        h = jnp.maximum(
            jnp.dot(x, w1_ref[...], preferred_element_type=jnp.float32)
            + b1_ref[...], 0.0)
        logits = (jnp.dot(h, w2_ref[...], preferred_element_type=jnp.float32)
                  + b2_ref[...])
        m = jnp.max(logits, axis=-1, keepdims=True)
        ex = jnp.exp(logits - m)
        probs = ex / jnp.sum(ex, axis=-1, keepdims=True)
        probs_s[pl.ds(t * tile, tile), :] = probs

        _, e0, _, _ = _top2(probs, tile, ne)
        flags = jnp.max((eidx == e0).astype(jnp.float32), axis=0,
                        keepdims=True)
        psum = jnp.sum(probs, axis=0, keepdims=True)
        first = t == 0
        c0 = jnp.where(first, flags, jnp.maximum(c0_s[...], flags))
        c0_s[...] = c0
        snap_s[pl.ds(t, 1), :] = c0
        sums_s[...] = jnp.where(first, psum, sums_s[...] + psum)

        @pl.when(t == tiles - 1)
        def _aux():
            mean = sums_s[...] * (1.0 / ntok)
            aux_ref[...] = jnp.sum(mean * jnp.log(mean * ne + 1e-09),
                                   axis=-1, keepdims=True)

        @pl.when(t < 2)
        def _zero_tail():
            zeros = jnp.zeros((tile, ne, cap), jnp.float32)
            abuf_d[slot] = zeros
            abuf_c[slot] = zeros

        @pl.when(t >= 2)
        def _reuse_wait():
            _acopy(abuf_d, disp_ref, slot, t - 2, sem_ad).wait()
            _acopy(abuf_c, comb_ref, slot, t - 2, sem_ac).wait()

        dlead, clead, _ = _lead_blocks(probs, c0, tile, ne)
        abuf_d[slot, :, :, 0:_LEAD] = dlead
        abuf_c[slot, :, :, 0:_LEAD] = clead
        _acopy(abuf_d, disp_ref, slot, t, sem_ad).start()
        _acopy(abuf_c, comb_ref, slot, t, sem_ac).start()
        probs_ref[...] = probs

    @pl.when(p == 1)
    def _phase_b():
        @pl.when(t == 0)
        def _drain_a():
            _acopy(abuf_d, disp_ref, (tiles - 2) % 2, tiles - 2, sem_ad).wait()
            _acopy(abuf_d, disp_ref, (tiles - 1) % 2, tiles - 1, sem_ad).wait()
            _acopy(abuf_c, comb_ref, (tiles - 2) % 2, tiles - 2, sem_ac).wait()
            _acopy(abuf_c, comb_ref, (tiles - 1) % 2, tiles - 1, sem_ac).wait()

        probs = probs_s[pl.ds(t * tile, tile), :]
        c0 = c0_s[...]
        snap = snap_s[pl.ds(t, 1), :]
        dlead, clead, e1 = _lead_blocks(probs, c0, tile, ne)
        wrong_e = jnp.logical_and(snap == 0.0, c0 == 1.0)
        wrong = jnp.max(jnp.sum(
            jnp.where(jnp.logical_and(eidx == e1, wrong_e), 1.0, 0.0),
            axis=-1))

        @pl.when(wrong > 0.0)
        def _fixup():
            fix_d[...] = dlead
            fix_c[...] = clead
            fd = pltpu.make_async_copy(
                fix_d, disp_ref.at[pl.ds(t * tile, tile), :, pl.ds(0, _LEAD)],
                sem_f)
            fc = pltpu.make_async_copy(
                fix_c, comb_ref.at[pl.ds(t * tile, tile), :, pl.ds(0, _LEAD)],
                sem_f)
            fd.start()
            fc.start()
            fd.wait()
            fc.wait()


def kernel(hidden_states, W1, b1, W2, b2):
    bb, ss, hh = hidden_states.shape
    ne = W2.shape[1]
    ntok = bb * ss
    cap = int(bb * ss * 1.5 * 2 / ne)
    x = hidden_states.reshape(ntok, hh)
    b1r = b1.reshape(1, hh)
    b2r = b2.reshape(1, ne)
    tile = 256
    tiles = ntok // tile

    body = functools.partial(_router_kernel, tile=tile, tiles=tiles,
                             ne=ne, cap=cap, ntok=ntok)

    disp, comb, probs, aux = pl.pallas_call(
        body,
        grid=(2, tiles),
        in_specs=[
            pl.BlockSpec((tile, hh), lambda p, t: (jnp.where(p == 0, t, 0), 0)),
            pl.BlockSpec((hh, hh), lambda p, t: (0, 0)),
            pl.BlockSpec((1, hh), lambda p, t: (0, 0)),
            pl.BlockSpec((hh, ne), lambda p, t: (0, 0)),
            pl.BlockSpec((1, ne), lambda p, t: (0, 0)),
        ],
        out_specs=[
            pl.BlockSpec(memory_space=pl.ANY),
            pl.BlockSpec(memory_space=pl.ANY),
            pl.BlockSpec((tile, ne),
                         lambda p, t: (jnp.where(p == 0, t, tiles - 1), 0)),
            pl.BlockSpec((1, 1), lambda p, t: (0, 0)),
        ],
        out_shape=[
            jax.ShapeDtypeStruct((ntok, ne, cap), jnp.float32),
            jax.ShapeDtypeStruct((ntok, ne, cap), jnp.float32),
            jax.ShapeDtypeStruct((ntok, ne), jnp.float32),
            jax.ShapeDtypeStruct((1, 1), jnp.float32),
        ],
        scratch_shapes=[
            pltpu.VMEM((ntok, ne), jnp.float32),
            pltpu.VMEM((1, ne), jnp.float32),
            pltpu.VMEM((tiles, ne), jnp.float32),
            pltpu.VMEM((1, ne), jnp.float32),
            pltpu.VMEM((2, tile, ne, cap), jnp.float32),
            pltpu.VMEM((2, tile, ne, cap), jnp.float32),
            pltpu.VMEM((tile, ne, _LEAD), jnp.float32),
            pltpu.VMEM((tile, ne, _LEAD), jnp.float32),
            pltpu.SemaphoreType.DMA((2,)),
            pltpu.SemaphoreType.DMA((2,)),
            pltpu.SemaphoreType.DMA,
        ],
        compiler_params=pltpu.CompilerParams(
            dimension_semantics=("arbitrary", "arbitrary")),
    )(x, W1, b1r, W2, b2r)

    return (disp.reshape(bb, ss, ne, cap),
            comb.reshape(bb, ss, ne, cap),
            probs.reshape(bb, ss, ne),
            aux.reshape(()))
